# Initial kernel scaffold; baseline (speedup 1.0000x reference)
import jax
import jax.numpy as jnp
from jax.experimental import pallas as pl


def kernel(coords, edge_index):
    raise NotImplementedError("write your pallas kernel here")


# speedup vs baseline: 4.6440x; 4.6440x over previous
import functools

import jax
import jax.numpy as jnp
from jax import lax
from jax.experimental import pallas as pl
from jax.experimental.pallas import tpu as pltpu
from jax.experimental.pallas import tpu_sc as plsc

_NC = 2
_NS = 16
_W = _NC * _NS


def _rsqrt(x):
    i = lax.bitcast_convert_type(x, jnp.int32)
    i = jnp.int32(0x5F3759DF) - (i >> 1)
    y = lax.bitcast_convert_type(i, jnp.float32)
    for _ in range(3):
        y = y * (1.5 - 0.5 * x * y * y)
    return y


@functools.partial(jax.jit, static_argnums=(2, 3))
def _run(coords4, eidx_flat, E, C):
    EPW = E // _W
    CH = EPW // C
    mesh = plsc.VectorSubcoreMesh(core_axis_name="c", subcore_axis_name="s")

    @functools.partial(
        pl.kernel,
        out_type=[
            jax.ShapeDtypeStruct((E,), jnp.float32),
            jax.ShapeDtypeStruct((E, 3), jnp.float32),
        ],
        mesh=mesh,
        scratch_types=[
            pltpu.VMEM((2 * C,), jnp.int32),
            pltpu.VMEM((2 * C, 4), jnp.float32),
            pltpu.VMEM((C,), jnp.float32),
            pltpu.VMEM((C, 3), jnp.float32),
            pltpu.SemaphoreType.DMA,
        ],
        compiler_params=pltpu.CompilerParams(
            use_tc_tiling_on_sc=False, needs_layout_passes=False
        ),
    )
    def k(coords_hbm, eidx_hbm, norm_hbm, vec_hbm, idx_v, rows_v, norm_v, vec_v, sem):
        wid = lax.axis_index("s") * _NC + lax.axis_index("c")

        def chunk_body(ci, carry):
            base = wid * EPW + ci * C
            pltpu.sync_copy(eidx_hbm.at[pl.ds(2 * base, 2 * C)], idx_v)
            pltpu.async_copy(coords_hbm.at[idx_v], rows_v, sem).wait()

            lanes = lax.iota(jnp.int32, 16)
            c0 = jnp.zeros((16,), jnp.int32)
            c1 = c0 + 1
            c2 = c0 + 2

            def grp(g, carry2):
                e_v = lanes + g * 16
                r_s = e_v * 2
                r_d = r_s + 1
                sx = plsc.load_gather(rows_v, [r_s, c0])
                sy = plsc.load_gather(rows_v, [r_s, c1])
                sz = plsc.load_gather(rows_v, [r_s, c2])
                dx = plsc.load_gather(rows_v, [r_d, c0])
                dy = plsc.load_gather(rows_v, [r_d, c1])
                dz = plsc.load_gather(rows_v, [r_d, c2])
                fx = sx - dx
                fy = sy - dy
                fz = sz - dz
                ss = fx * fx + fy * fy + fz * fz + 1e-6
                r = _rsqrt(ss)
                nrm = ss * r
                inv = 1.0 / (nrm + 1.0)
                norm_v[pl.ds(g * 16, 16)] = nrm
                plsc.store_scatter(vec_v, [e_v, c0], fx * inv)
                plsc.store_scatter(vec_v, [e_v, c1], fy * inv)
                plsc.store_scatter(vec_v, [e_v, c2], fz * inv)
                return carry2

            lax.fori_loop(0, C // 16, grp, 0)
            pltpu.sync_copy(norm_v, norm_hbm.at[pl.ds(base, C)])
            pltpu.sync_copy(vec_v, vec_hbm.at[pl.ds(base, C)])
            return carry

        lax.fori_loop(0, CH, chunk_body, 0)

    return k(coords4, eidx_flat)


def kernel(coords, edge_index):
    E = edge_index.shape[0]
    coords4 = jnp.pad(coords.astype(jnp.float32), ((0, 0), (0, 1)))
    eidx_flat = edge_index.astype(jnp.int32).reshape(-1)
    C = 4000
    norm_flat, vecs = _run(coords4, eidx_flat, E, C)
    return norm_flat[:, None], vecs


# speedup vs baseline: 73.2082x; 15.7640x over previous
import functools

import jax
import jax.numpy as jnp
from jax import lax
from jax.experimental import pallas as pl
from jax.experimental.pallas import tpu as pltpu
from jax.experimental.pallas import tpu_sc as plsc

_NC = 2
_NS = 16
_W = _NC * _NS
_CB = 11


def _rsqrt(x):
    i = lax.bitcast_convert_type(x, jnp.int32)
    i = jnp.int32(0x5F3759DF) - (i >> 1)
    y = lax.bitcast_convert_type(i, jnp.float32)
    for _ in range(3):
        y = y * (1.5 - 0.5 * x * y * y)
    return y


@functools.partial(jax.jit, static_argnums=(2,))
def _run(coords4, eidx2, E):
    NB = E // 128
    per_w = NB // _W
    n_extra = NB - per_w * _W
    NCH = per_w // _CB
    assert NCH * _CB == per_w and NCH % 2 == 0 and n_extra < _W

    mesh = plsc.VectorSubcoreMesh(core_axis_name="c", subcore_axis_name="s")

    @functools.partial(
        pl.kernel,
        out_type=[
            jax.ShapeDtypeStruct((E,), jnp.float32),
            jax.ShapeDtypeStruct((NB * 512,), jnp.float32),
        ],
        mesh=mesh,
        scratch_types=[
            pltpu.VMEM((2 * _CB, 128), jnp.int32),
            pltpu.VMEM((2 * _CB, 128), jnp.int32),
            pltpu.VMEM((_CB * 256, 4), jnp.float32),
            pltpu.VMEM((_CB * 256, 4), jnp.float32),
            pltpu.VMEM((_CB * 512,), jnp.float32),
            pltpu.VMEM((_CB * 512,), jnp.float32),
            pltpu.VMEM((_CB * 128,), jnp.float32),
            pltpu.VMEM((_CB * 128,), jnp.float32),
            pltpu.SemaphoreType.DMA,
            pltpu.SemaphoreType.DMA,
            pltpu.SemaphoreType.DMA,
            pltpu.SemaphoreType.DMA,
        ],
        compiler_params=pltpu.CompilerParams(
            use_tc_tiling_on_sc=False, needs_layout_passes=False
        ),
    )
    def k(coords_hbm, eidx_hbm, norm_hbm, vec_hbm,
          idx_a, idx_b, rows_a, rows_b, vec_a, vec_b, nrm_a, nrm_b,
          gsem_a, gsem_b, osem_a, osem_b):
        wid = lax.axis_index("s") * _NC + lax.axis_index("c")
        start_w = wid * per_w + jnp.minimum(wid, n_extra)

        lanes = lax.iota(jnp.int32, 16)
        c0 = jnp.zeros((16,), jnp.int32)
        c1 = c0 + 1
        c2 = c0 + 2

        def load_idx(t, idx_v):
            gb = start_w + t * _CB
            pltpu.sync_copy(eidx_hbm.at[pl.ds(2 * gb, 2 * _CB)], idx_v)

        def fire_gathers(idx_v, rows_v, gsem):
            def fire(r, c):
                pltpu.async_copy(
                    coords_hbm.at[idx_v.at[r]],
                    rows_v.at[pl.ds(r * 128, 128)],
                    gsem,
                )
                return c

            lax.fori_loop(0, 2 * _CB, fire, 0)

        def drain_gathers(idx_v, rows_v, gsem):
            def drain(r, c):
                pltpu.make_async_copy(
                    coords_hbm.at[idx_v.at[r]],
                    rows_v.at[pl.ds(r * 128, 128)],
                    gsem,
                ).wait()
                return c

            lax.fori_loop(0, 2 * _CB, drain, 0)

        def compute(rows_v, vec_v, nrm_v, n_blocks):
            def grp(gi, c):
                b = gi >> 3
                m = gi & 7
                row_s = lanes + (b * 256 + m * 16)
                row_d = row_s + 128
                sx = plsc.load_gather(rows_v, [row_s, c0])
                sy = plsc.load_gather(rows_v, [row_s, c1])
                sz = plsc.load_gather(rows_v, [row_s, c2])
                dx = plsc.load_gather(rows_v, [row_d, c0])
                dy = plsc.load_gather(rows_v, [row_d, c1])
                dz = plsc.load_gather(rows_v, [row_d, c2])
                fx = sx - dx
                fy = sy - dy
                fz = sz - dz
                ss = fx * fx + fy * fy + fz * fz + 1e-6
                r = _rsqrt(ss)
                nrm = ss * r
                inv = 1.0 / (nrm + 1.0)
                o = b * 512 + m * 16
                vec_v[pl.ds(o, 16)] = fx * inv
                vec_v[pl.ds(o + 128, 16)] = fy * inv
                vec_v[pl.ds(o + 256, 16)] = fz * inv
                nrm_v[pl.ds(b * 128 + m * 16, 16)] = nrm
                return c

            lax.fori_loop(0, 8 * n_blocks, grp, 0)

        def out_dma(t, vec_v, nrm_v, osem):
            gb = start_w + t * _CB
            pltpu.async_copy(vec_v, vec_hbm.at[pl.ds(512 * gb, 512 * _CB)], osem)
            pltpu.async_copy(nrm_v, norm_hbm.at[pl.ds(128 * gb, 128 * _CB)], osem)

        def out_wait(t, vec_v, nrm_v, osem):
            gb = start_w + t * _CB
            pltpu.make_async_copy(
                vec_v, vec_hbm.at[pl.ds(512 * gb, 512 * _CB)], osem
            ).wait()
            pltpu.make_async_copy(
                nrm_v, norm_hbm.at[pl.ds(128 * gb, 128 * _CB)], osem
            ).wait()

        load_idx(0, idx_a)
        fire_gathers(idx_a, rows_a, gsem_a)

        def pair(u, c):
            t0 = 2 * u
            load_idx(t0 + 1, idx_b)
            fire_gathers(idx_b, rows_b, gsem_b)
            drain_gathers(idx_a, rows_a, gsem_a)
            compute(rows_a, vec_a, nrm_a, _CB)
            out_dma(t0, vec_a, nrm_a, osem_a)
            @pl.when(u + 1 < NCH // 2)
            def _():
                load_idx(t0 + 2, idx_a)
                fire_gathers(idx_a, rows_a, gsem_a)
            drain_gathers(idx_b, rows_b, gsem_b)
            compute(rows_b, vec_b, nrm_b, _CB)
            out_dma(t0 + 1, vec_b, nrm_b, osem_b)
            out_wait(t0, vec_a, nrm_a, osem_a)
            out_wait(t0 + 1, vec_b, nrm_b, osem_b)
            return c

        lax.fori_loop(0, NCH // 2, pair, 0)

        @pl.when(wid < n_extra)
        def _():
            gb = start_w + per_w
            pltpu.sync_copy(eidx_hbm.at[pl.ds(2 * gb, 2)],
                            idx_a.at[pl.ds(0, 2)])
            fire2 = pltpu.async_copy(
                coords_hbm.at[idx_a.at[0]], rows_a.at[pl.ds(0, 128)], gsem_a
            )
            fire3 = pltpu.async_copy(
                coords_hbm.at[idx_a.at[1]], rows_a.at[pl.ds(128, 128)], gsem_a
            )
            fire2.wait()
            fire3.wait()
            compute(rows_a, vec_a, nrm_a, 1)
            pltpu.sync_copy(vec_a.at[pl.ds(0, 512)],
                            vec_hbm.at[pl.ds(512 * gb, 512)])
            pltpu.sync_copy(nrm_a.at[pl.ds(0, 128)],
                            norm_hbm.at[pl.ds(128 * gb, 128)])

    return k(coords4, eidx2)


def kernel(coords, edge_index):
    E = edge_index.shape[0]
    NB = E // 128
    coords4 = jnp.pad(coords, ((0, 0), (0, 1)))
    eidx2 = edge_index.reshape(NB, 128, 2).transpose(0, 2, 1).reshape(2 * NB, 128)
    norm_flat, vec_flat = _run(coords4, eidx2, E)
    vecs = vec_flat.reshape(NB, 4, 128).transpose(0, 2, 1).reshape(E, 4)[:, :3]
    return norm_flat[:, None], vecs
